# baseline (speedup 1.0000x reference)
import functools

import jax
import jax.numpy as jnp
from jax import lax
from jax.experimental import pallas as pl
from jax.experimental.pallas import tpu as pltpu
from jax.experimental.pallas import tpu_sc as plsc

MAXH = 3
D = 128
NC, NS, L = 2, 16, 16
NW = NC * NS
CHUNK = 256


def _sc_lookup(idx_flat, table, m_total):
    m_per_w = m_total // NW
    n_chunks = m_per_w // CHUNK
    mesh = plsc.VectorSubcoreMesh(core_axis_name="c", subcore_axis_name="s")

    @functools.partial(
        pl.kernel,
        out_type=jax.ShapeDtypeStruct((m_total * D,), jnp.float32),
        mesh=mesh,
        compiler_params=pltpu.CompilerParams(needs_layout_passes=False),
        scratch_types=[
            pltpu.VMEM(((MAXH + 1) * D,), jnp.float32),
            pltpu.VMEM((CHUNK,), jnp.int32),
            pltpu.VMEM((CHUNK * D,), jnp.float32),
        ],
    )
    def k(idx_hbm, table_hbm, out_hbm, table_v, idx_v, rows_v):
        wid = lax.axis_index("s") * NC + lax.axis_index("c")
        base = wid * m_per_w
        pltpu.sync_copy(table_hbm, table_v)
        lane = lax.iota(jnp.int32, L)

        def chunk_body(g, _):
            off = base + g * CHUNK
            pltpu.sync_copy(idx_hbm.at[pl.ds(off, CHUNK)], idx_v)

            def group_body(t, _):
                row0 = t * L
                ids = idx_v[pl.ds(row0, L)]
                addr = jnp.clip(ids, 0, MAXH) * D
                dst0 = (lane + row0) * D
                for c in range(D):
                    val = plsc.load_gather(table_v, [addr + c])
                    plsc.store_scatter(rows_v, [dst0 + c], val)
                return 0

            lax.fori_loop(0, CHUNK // L, group_body, 0, unroll=False)
            pltpu.sync_copy(rows_v, out_hbm.at[pl.ds(off * D, CHUNK * D)])
            return 0

        lax.fori_loop(0, n_chunks, chunk_body, 0, unroll=False)

    return k(idx_flat, table)


def kernel(hop_distances, hop_embedding):
    b, n = hop_distances.shape
    m_total = b * n
    idx_flat = hop_distances.reshape(m_total).astype(jnp.int32)
    table_flat = hop_embedding.astype(jnp.float32).reshape(-1)
    out = _sc_lookup(idx_flat, table_flat, m_total)
    return out.reshape(b, n, D)
